# Initial kernel scaffold; baseline (speedup 1.0000x reference)
import jax
import jax.numpy as jnp
from jax.experimental import pallas as pl


def kernel(x, table):
    raise NotImplementedError("write your pallas kernel here")


# speedup vs baseline: 2.7718x; 2.7718x over previous
import jax
import jax.numpy as jnp
from jax.experimental import pallas as pl
from jax.experimental.pallas import tpu as pltpu

_BLK = 4096
_UNROLL = 8


def _gather_kernel(ids_ref, table_ref, out_ref, ids_smem, sem):
    cp = pltpu.make_async_copy(ids_ref.at[0], ids_smem, sem)
    cp.start()
    cp.wait()

    def body(o, _):
        base = o * _UNROLL
        for k in range(_UNROLL):
            idx = ids_smem[0, base + k]
            out_ref[base + k, 0] = table_ref[idx, 0]
        return 0

    jax.lax.fori_loop(0, _BLK // _UNROLL, body, 0)


def kernel(x, table):
    B, T = x.shape
    V, D = table.shape
    N = B * T
    ids = x.reshape(N).astype(jnp.int32)

    blk = _BLK
    n_pad = (-N) % blk
    if n_pad:
        ids = jnp.pad(ids, (0, n_pad))
    n_tot = N + n_pad
    nb = n_tot // blk

    ids3 = ids.reshape(nb, 1, blk)
    table3 = table.reshape(V, 1, D)

    table_bytes = V * D * jnp.dtype(table.dtype).itemsize
    vmem_limit = int(table_bytes + 4 * blk * D * 4 + (8 << 20))

    out = pl.pallas_call(
        _gather_kernel,
        out_shape=jax.ShapeDtypeStruct((n_tot, 1, D), table.dtype),
        grid_spec=pltpu.PrefetchScalarGridSpec(
            num_scalar_prefetch=0,
            grid=(nb,),
            in_specs=[
                pl.BlockSpec((1, 1, blk), lambda i: (i, 0, 0)),
                pl.BlockSpec((V, 1, D), lambda i: (0, 0, 0),
                             pipeline_mode=pl.Buffered(1)),
            ],
            out_specs=pl.BlockSpec((blk, 1, D), lambda i: (i, 0, 0)),
            scratch_shapes=[
                pltpu.SMEM((1, blk), jnp.int32),
                pltpu.SemaphoreType.DMA,
            ],
        ),
        compiler_params=pltpu.CompilerParams(
            dimension_semantics=("parallel",),
            vmem_limit_bytes=vmem_limit,
        ),
        cost_estimate=pl.CostEstimate(
            flops=0,
            transcendentals=0,
            bytes_accessed=table_bytes + n_tot * D * 4 + n_tot * 4,
        ),
    )(ids3, table3)

    return out[:N].reshape(B, T, D)


# speedup vs baseline: 3.0699x; 1.1075x over previous
import jax
import jax.numpy as jnp
from jax.experimental import pallas as pl
from jax.experimental.pallas import tpu as pltpu

_BLK = 4096
_UNROLL = 16
_NCHUNK = 8


def _gather_kernel(ids_ref, table_ref, out_ref, ids_smem, slab, sem_ids,
                   sem_out):
    cp = pltpu.make_async_copy(ids_ref.at[0], ids_smem, sem_ids)
    cp.start()
    cp.wait()

    chunk = _BLK // _NCHUNK
    copies = []
    for c in range(_NCHUNK):
        def body(o, _, c=c):
            base = c * chunk + o * _UNROLL
            rows = [table_ref[ids_smem[0, base + k], 0]
                    for k in range(_UNROLL)]
            for k in range(_UNROLL):
                slab[base + k, 0] = rows[k]
            return 0

        jax.lax.fori_loop(0, chunk // _UNROLL, body, 0)
        cpc = pltpu.make_async_copy(
            slab.at[pl.ds(c * chunk, chunk), 0],
            out_ref.at[pl.ds(c * chunk, chunk)],
            sem_out)
        cpc.start()
        copies.append(cpc)
    for cpc in copies:
        cpc.wait()


def kernel(x, table):
    B, T = x.shape
    V, D = table.shape
    N = B * T
    ids = x.reshape(N).astype(jnp.int32)

    blk = _BLK
    n_pad = (-N) % blk
    if n_pad:
        ids = jnp.pad(ids, (0, n_pad))
    n_tot = N + n_pad
    nb = n_tot // blk

    ids3 = ids.reshape(nb, 1, blk)
    table3 = table.reshape(V, 1, D)

    table_bytes = V * D * jnp.dtype(table.dtype).itemsize
    vmem_limit = int(table_bytes + 6 * blk * D * 4 + (8 << 20))

    out = pl.pallas_call(
        _gather_kernel,
        out_shape=jax.ShapeDtypeStruct((n_tot, D), table.dtype),
        grid_spec=pltpu.PrefetchScalarGridSpec(
            num_scalar_prefetch=0,
            grid=(nb,),
            in_specs=[
                pl.BlockSpec((1, 1, blk), lambda i: (i, 0, 0)),
                pl.BlockSpec((V, 1, D), lambda i: (0, 0, 0),
                             pipeline_mode=pl.Buffered(1)),
            ],
            out_specs=pl.BlockSpec((blk, D), lambda i: (i, 0)),
            scratch_shapes=[
                pltpu.SMEM((1, blk), jnp.int32),
                pltpu.VMEM((blk, 1, D), table.dtype),
                pltpu.SemaphoreType.DMA,
                pltpu.SemaphoreType.DMA,
            ],
        ),
        compiler_params=pltpu.CompilerParams(
            dimension_semantics=("parallel",),
            vmem_limit_bytes=vmem_limit,
        ),
        cost_estimate=pl.CostEstimate(
            flops=0,
            transcendentals=0,
            bytes_accessed=table_bytes + n_tot * D * 4 + n_tot * 4,
        ),
    )(ids3, table3)

    if n_pad:
        out = out[:N]
    return out.reshape(B, T, D)


# speedup vs baseline: 5.5677x; 1.8136x over previous
import jax
import jax.numpy as jnp
from jax.experimental import pallas as pl
from jax.experimental.pallas import tpu as pltpu

_BLK = 4096
_UNROLL = 16


def _gather_kernel(ids_ref, table_ref, out_hbm, ids_smem, slab_a, slab_b,
                   sem_ids, sems):
    i = pl.program_id(0)
    nb = pl.num_programs(0)
    blk = slab_a.shape[0]

    def out_copy(slab, step, slot):
        return pltpu.make_async_copy(
            slab.at[:, 0],
            out_hbm.at[pl.ds(step * blk, blk), :],
            sems.at[slot])

    cp = pltpu.make_async_copy(ids_ref.at[0], ids_smem, sem_ids)
    cp.start()
    cp.wait()

    par = jax.lax.rem(i, 2)

    def run(slab, slot):
        @pl.when(i >= 2)
        def _():
            out_copy(slab, i - 2, slot).wait()

        def body(o, _):
            base = o * _UNROLL
            rows = [table_ref[ids_smem[0, base + k], 0]
                    for k in range(_UNROLL)]
            for k in range(_UNROLL):
                slab[base + k, 0] = rows[k]
            return 0

        jax.lax.fori_loop(0, blk // _UNROLL, body, 0)
        out_copy(slab, i, slot).start()

    @pl.when(par == 0)
    def _():
        run(slab_a, 0)

    @pl.when(par == 1)
    def _():
        run(slab_b, 1)

    @pl.when(i == nb - 1)
    def _():
        out_copy(slab_a, 0, par).wait()
        if nb > 1:
            out_copy(slab_a, 0, 1 - par).wait()


def kernel(x, table):
    B, T = x.shape
    V, D = table.shape
    N = B * T
    ids = x.reshape(N).astype(jnp.int32)

    blk = _BLK
    n_pad = (-N) % blk
    if n_pad:
        ids = jnp.pad(ids, (0, n_pad))
    n_tot = N + n_pad
    nb = n_tot // blk

    ids3 = ids.reshape(nb, 1, blk)
    table3 = table.reshape(V, 1, D)

    table_bytes = V * D * jnp.dtype(table.dtype).itemsize
    vmem_limit = int(table_bytes + 3 * blk * D * 4 + (8 << 20))

    out = pl.pallas_call(
        _gather_kernel,
        out_shape=jax.ShapeDtypeStruct((n_tot, D), table.dtype),
        grid_spec=pltpu.PrefetchScalarGridSpec(
            num_scalar_prefetch=0,
            grid=(nb,),
            in_specs=[
                pl.BlockSpec((1, 1, blk), lambda i: (i, 0, 0)),
                pl.BlockSpec((V, 1, D), lambda i: (0, 0, 0),
                             pipeline_mode=pl.Buffered(1)),
            ],
            out_specs=pl.BlockSpec(memory_space=pl.ANY),
            scratch_shapes=[
                pltpu.SMEM((1, blk), jnp.int32),
                pltpu.VMEM((blk, 1, D), table.dtype),
                pltpu.VMEM((blk, 1, D), table.dtype),
                pltpu.SemaphoreType.DMA,
                pltpu.SemaphoreType.DMA((2,)),
            ],
        ),
        compiler_params=pltpu.CompilerParams(
            dimension_semantics=("arbitrary",),
            vmem_limit_bytes=vmem_limit,
        ),
        cost_estimate=pl.CostEstimate(
            flops=0,
            transcendentals=0,
            bytes_accessed=table_bytes + n_tot * D * 4 + n_tot * 4,
        ),
    )(ids3, table3)

    if n_pad:
        out = out[:N]
    return out.reshape(B, T, D)
